# baseline (speedup 1.0000x reference)
import functools

import jax
import jax.numpy as jnp
from jax import lax
from jax.experimental import pallas as pl
from jax.experimental.pallas import tpu as pltpu
from jax.experimental.pallas import tpu_sc as plsc

B = 8
N0 = 2048
K = 16
DM = 32

_SC_CORES = 2
_SC_SUBCORES = 16
_SC_WORKERS = _SC_CORES * _SC_SUBCORES


def _sa_td(c):
    return ((16 + c + 127) // 128) * 128


def _lin(x, w, b=None):
    y = jnp.dot(x, w, preferred_element_type=jnp.float32)
    if b is not None:
        y = y + b
    return y


def _lin3(x, w, b=None):
    y = x[:, 0:1] * w[0:1, :] + x[:, 1:2] * w[1:2, :] + x[:, 2:3] * w[2:3, :]
    if b is not None:
        y = y + b
    return y


def _wspec(shape):
    nd = len(shape)
    return pl.BlockSpec(shape, lambda *_: (0,) * nd)


def _rep_rows(x, k):
    m, c = x.shape
    return jnp.broadcast_to(x[:, None, :], (m, k, c)).reshape(m * k, c)


def _gather_rows(table, idx):
    v, d = table.shape
    m = idx.shape[0]
    b_per_w = m // _SC_WORKERS
    ch = min(128, b_per_w)
    n_ch = b_per_w // ch
    gsz = max(1, min(n_ch, (384 * 1024) // (ch * d * 4)))
    while n_ch % gsz:
        gsz -= 1
    ngrp = n_ch // gsz
    mesh = plsc.VectorSubcoreMesh(core_axis_name="c", subcore_axis_name="s")

    @functools.partial(
        pl.kernel,
        out_type=jax.ShapeDtypeStruct((m, d), jnp.float32),
        mesh=mesh,
        scratch_types=[
            pltpu.VMEM((n_ch, ch), jnp.int32),
            pltpu.VMEM((gsz * ch, d), jnp.float32),
            pltpu.SemaphoreType.DMA,
        ],
    )
    def gk(table_hbm, idx_hbm, out_hbm, idx_v, rows_v, sem):
        wid = lax.axis_index("s") * _SC_CORES + lax.axis_index("c")
        base = wid * b_per_w
        pltpu.sync_copy(idx_hbm.at[pl.ds(wid * n_ch, n_ch)], idx_v)

        def group(gi, carry):
            handles = [
                pltpu.async_copy(
                    table_hbm.at[idx_v.at[gi * gsz + u]],
                    rows_v.at[pl.ds(u * ch, ch)], sem)
                for u in range(gsz)
            ]
            for h in handles:
                h.wait()
            pltpu.sync_copy(
                rows_v, out_hbm.at[pl.ds(base + gi * (gsz * ch), gsz * ch)])
            return carry

        if ngrp == 1:
            group(0, 0)
        else:
            lax.fori_loop(0, ngrp, group, 0)

    return gk(table, idx.reshape(m // ch, ch))


def _embed_body(x_ref, e1w, e1b, e2w, e2b, f1w, f1b, wq, wk, wv,
                h_ref, q_ref, tab_ref):
    xb = x_ref[0]
    h1 = jax.nn.relu(_lin3(xb, e1w[...], e1b[...]))
    h = _lin(h1, e2w[...], e2b[...])
    x1 = _lin(h, f1w[...], f1b[...])
    h_ref[0] = h
    q_ref[0] = _lin(x1, wq[...])
    tab_ref[0, :, 0:3] = xb
    tab_ref[0, :, 16:48] = _lin(x1, wk[...])
    tab_ref[0, :, 48:80] = _lin(x1, wv[...])


def _embed(x, e1w, e1b, e2w, e2b, f1w, f1b, wq, wk, wv):
    br = 256
    grid = (B, N0 // br)
    ws = [e1w, e1b, e2w, e2b, f1w, f1b, wq, wk, wv]
    return pl.pallas_call(
        _embed_body,
        grid=grid,
        in_specs=[pl.BlockSpec((1, br, 3), lambda b, r: (b, r, 0))]
        + [_wspec(w.shape) for w in ws],
        out_specs=[
            pl.BlockSpec((1, br, DM), lambda b, r: (b, r, 0)),
            pl.BlockSpec((1, br, DM), lambda b, r: (b, r, 0)),
            pl.BlockSpec((1, br, 128), lambda b, r: (b, r, 0)),
        ],
        out_shape=[
            jax.ShapeDtypeStruct((B, N0, DM), jnp.float32),
            jax.ShapeDtypeStruct((B, N0, DM), jnp.float32),
            jax.ShapeDtypeStruct((B, N0, 128), jnp.float32),
        ],
        compiler_params=pltpu.CompilerParams(
            dimension_semantics=("parallel", "arbitrary")),
    )(x, *ws)


def _knn_body(nd, k, src_ref, dstt_ref, idx_ref):
    b = pl.program_id(0)
    src = src_ref[0]
    br = src.shape[0]
    dt = dstt_ref[0]
    d = jnp.zeros((br, nd), jnp.float32)
    for c in range(3):
        diff = src[:, c:c + 1] - dt[c:c + 1, :]
        d = d + diff * diff
    iota = lax.broadcasted_iota(jnp.int32, (br, nd), 1)
    off = b * nd
    for j in range(k):
        m = jnp.min(d, axis=1, keepdims=True)
        jm = jnp.min(jnp.where(d <= m, iota, nd), axis=1, keepdims=True)
        idx_ref[0, :, j:j + 1] = jm + off
        d = jnp.where(iota == jm, jnp.float32(jnp.inf), d)


def _knn(src, dstt, k, br):
    _, ns, _ = src.shape
    nd = dstt.shape[2]
    grid = (B, ns // br)
    out = pl.pallas_call(
        functools.partial(_knn_body, nd, k),
        grid=grid,
        in_specs=[
            pl.BlockSpec((1, br, 3), lambda b, r: (b, r, 0)),
            pl.BlockSpec((1, 3, nd), lambda b, r: (b, 0, 0)),
        ],
        out_specs=pl.BlockSpec((1, br, k), lambda b, r: (b, r, 0)),
        out_shape=jax.ShapeDtypeStruct((B, ns, k), jnp.int32),
        compiler_params=pltpu.CompilerParams(
            dimension_semantics=("parallel", "arbitrary")),
    )(src, dstt)
    return out.reshape(-1)


def _fps_body(n, npoint, xyz_ref, xyzt_ref, nx_ref):
    nl = n // 8
    flat = (lax.broadcasted_iota(jnp.int32, (8, nl), 0) * nl
            + lax.broadcasted_iota(jnp.int32, (8, nl), 1))

    def step(t, carry):
        dist, far = carry
        c = xyz_ref[0, pl.ds(far, 1), :]
        nx_ref[0, pl.ds(t, 1), :] = c
        d = jnp.zeros((8, nl), jnp.float32)
        for cc in range(3):
            diff = xyzt_ref[0, cc] - c[0, cc]
            d = d + diff * diff
        dist = jnp.minimum(dist, d)
        m = jnp.max(dist)
        far2 = jnp.min(jnp.where(dist >= m, flat, n))
        return dist, far2

    init = (jnp.full((8, nl), 1e10, jnp.float32), jnp.int32(0))
    lax.fori_loop(0, npoint, step, init)


def _fps(xyz, xyzt, npoint):
    n = xyz.shape[1]
    xyzr = xyzt.reshape(B, 3, 8, n // 8)
    return pl.pallas_call(
        functools.partial(_fps_body, n, npoint),
        grid=(B,),
        in_specs=[
            pl.BlockSpec((1, n, 3), lambda b: (b, 0, 0)),
            pl.BlockSpec((1, 3, 8, n // 8), lambda b: (b, 0, 0, 0)),
        ],
        out_specs=pl.BlockSpec((1, npoint, 3), lambda b: (b, 0, 0)),
        out_shape=jax.ShapeDtypeStruct((B, npoint, 3), jnp.float32),
        compiler_params=pltpu.CompilerParams(
            dimension_semantics=("parallel",)),
    )(xyz, xyzr)


def _tf_pre_body(pts_ref, xyz_ref, f1w, f1b, wq, wk, wv, q_ref, tab_ref):
    p = pts_ref[0]
    x1 = _lin(p, f1w[...], f1b[...])
    q_ref[0] = _lin(x1, wq[...])
    tab_ref[0, :, 0:3] = xyz_ref[0]
    tab_ref[0, :, 16:48] = _lin(x1, wk[...])
    tab_ref[0, :, 48:80] = _lin(x1, wv[...])


def _tf_pre(pts, xyz, f1w, f1b, wq, wk, wv):
    _, ns, c = pts.shape
    ws = [f1w, f1b, wq, wk, wv]
    return pl.pallas_call(
        _tf_pre_body,
        grid=(B,),
        in_specs=[
            pl.BlockSpec((1, ns, c), lambda b: (b, 0, 0)),
            pl.BlockSpec((1, ns, 3), lambda b: (b, 0, 0)),
        ] + [_wspec(w.shape) for w in ws],
        out_specs=[
            pl.BlockSpec((1, ns, DM), lambda b: (b, 0, 0)),
            pl.BlockSpec((1, ns, 128), lambda b: (b, 0, 0)),
        ],
        out_shape=[
            jax.ShapeDtypeStruct((B, ns, DM), jnp.float32),
            jax.ShapeDtypeStruct((B, ns, 128), jnp.float32),
        ],
        compiler_params=pltpu.CompilerParams(
            dimension_semantics=("parallel",)),
    )(pts, xyz, *ws)


def _tf_post_body(k, c, emit_tab, g_ref, xyz_ref, q_ref, pre_ref,
                  d1w, d1b, d2w, d2b, g1w, g1b, g2w, g2b, f2w, f2b,
                  *out_refs):
    out_ref = out_refs[0]
    g = g_ref[0]
    xyz = xyz_ref[0]
    br = xyz.shape[0]
    pos = _rep_rows(xyz, k) - g[:, 0:3]
    ph = jax.nn.relu(_lin3(pos, d1w[...], d1b[...]))
    pos_enc = _lin(ph, d2w[...], d2b[...])
    t = _rep_rows(q_ref[0], k) - g[:, 16:48] + pos_enc
    a1 = jax.nn.relu(_lin(t, g1w[...], g1b[...]))
    a2 = _lin(a1, g2w[...], g2b[...]) * jnp.float32(1.0 / (32.0 ** 0.5))
    a3 = a2.reshape(br, k, DM)
    mx = jnp.max(a3, axis=1)
    e = jnp.exp(a3 - mx[:, None, :])
    s = jnp.sum(e, axis=1)
    w3 = e / s[:, None, :]
    vp = (g[:, 48:80] + pos_enc).reshape(br, k, DM)
    res = jnp.sum(w3 * vp, axis=1)
    out = _lin(res, f2w[...], f2b[...]) + pre_ref[0]
    out_ref[0] = out
    if emit_tab:
        out_refs[1][0, :, 0:3] = xyz
        out_refs[1][0, :, 16:16 + c] = out


def _tf_post(g, xyz, q, pre, k, wlist, emit_tab):
    _, ns, c = pre.shape
    br = min(64, ns)
    grid = (B, ns // br)
    tab_d = _sa_td(c)
    out_specs = [pl.BlockSpec((1, br, c), lambda b, r: (b, r, 0))]
    out_shape = [jax.ShapeDtypeStruct((B, ns, c), jnp.float32)]
    if emit_tab:
        out_specs.append(pl.BlockSpec((1, br, tab_d), lambda b, r: (b, r, 0)))
        out_shape.append(jax.ShapeDtypeStruct((B, ns, tab_d), jnp.float32))
    outs = pl.pallas_call(
        functools.partial(_tf_post_body, k, c, emit_tab),
        grid=grid,
        in_specs=[
            pl.BlockSpec((1, br * k, 128), lambda b, r: (b, r, 0)),
            pl.BlockSpec((1, br, 3), lambda b, r: (b, r, 0)),
            pl.BlockSpec((1, br, DM), lambda b, r: (b, r, 0)),
            pl.BlockSpec((1, br, c), lambda b, r: (b, r, 0)),
        ] + [_wspec(w.shape) for w in wlist],
        out_specs=out_specs,
        out_shape=out_shape,
        compiler_params=pltpu.CompilerParams(
            dimension_semantics=("parallel", "arbitrary")),
    )(g, xyz, q, pre, *wlist)
    return outs if emit_tab else (outs[0], None)


def _sa_post_body(k, c, g_ref, nx_ref,
                  w1x, w1p, b1, ga1, be1, w2, b2, ga2, be2, out_ref):
    g = g_ref[0]
    nx = nx_ref[0]
    br = nx.shape[0]
    ch = out_ref.shape[2]
    inv = 1.0 / jnp.sqrt(jnp.float32(1.0 + 1e-5))
    xyzn = g[:, 0:3] - _rep_rows(nx, k)
    h = _lin3(xyzn, w1x[...]) + _lin(g[:, 16:16 + c], w1p[...]) + b1[...]
    h = jax.nn.relu(h * inv * ga1[...] + be1[...])
    h = _lin(h, w2[...], b2[...])
    h = jax.nn.relu(h * inv * ga2[...] + be2[...])
    out_ref[0] = jnp.max(h.reshape(br, k, ch), axis=1)


def _sa_post(g, nx, k, c, ch, wlist):
    npnt = nx.shape[1]
    br = min(64, npnt)
    grid = (B, npnt // br)
    tab_d = _sa_td(c)
    return pl.pallas_call(
        functools.partial(_sa_post_body, k, c),
        grid=grid,
        in_specs=[
            pl.BlockSpec((1, br * k, tab_d), lambda b, r: (b, r, 0)),
            pl.BlockSpec((1, br, 3), lambda b, r: (b, r, 0)),
        ] + [_wspec(w.shape) for w in wlist],
        out_specs=pl.BlockSpec((1, br, ch), lambda b, r: (b, r, 0)),
        out_shape=jax.ShapeDtypeStruct((B, npnt, ch), jnp.float32),
        compiler_params=pltpu.CompilerParams(
            dimension_semantics=("parallel", "arbitrary")),
    )(g, nx, *wlist)


def _fc2_body(p_ref, w1, b1, w2, b2, w3, b3, out_ref):
    r = jax.nn.relu(_lin(p_ref[...], w1[...], b1[...]))
    r = jax.nn.relu(_lin(r, w2[...], b2[...]))
    out_ref[...] = _lin(r, w3[...], b3[...])


def _fc2(pts, wlist):
    m = pts.shape[0]
    return pl.pallas_call(
        _fc2_body,
        out_shape=jax.ShapeDtypeStruct((m, DM), jnp.float32),
    )(pts, *wlist)


def _row(v):
    return v.reshape(1, -1)


def kernel(x, params):
    p = params
    xyz = x
    xyzt = jnp.swapaxes(xyz, 1, 2)

    e1, e2 = p['fc1']
    t1 = p['transformer1']
    h, q, tab = _embed(
        x, e1['w'], _row(e1['b']), e2['w'], _row(e2['b']),
        t1['fc1']['w'], _row(t1['fc1']['b']),
        t1['w_qs']['w'], t1['w_ks']['w'], t1['w_vs']['w'])

    def tf_wlist(tp):
        return [
            tp['fc_delta'][0]['w'], _row(tp['fc_delta'][0]['b']),
            tp['fc_delta'][1]['w'], _row(tp['fc_delta'][1]['b']),
            tp['fc_gamma'][0]['w'], _row(tp['fc_gamma'][0]['b']),
            tp['fc_gamma'][1]['w'], _row(tp['fc_gamma'][1]['b']),
            tp['fc2']['w'], _row(tp['fc2']['b']),
        ]

    n = N0
    idx = _knn(xyz, xyzt, K, 64)
    g = _gather_rows(tab.reshape(B * n, 128), idx).reshape(B, n * K, 128)
    points, sa_tab = _tf_post(g, xyz, q, h, K, tf_wlist(t1), True)

    for i in range(4):
        npnt = N0 // 4 ** (i + 1)
        ch = DM * 2 ** (i + 1)
        c = ch // 2
        td = p['td'][i]['mlps']
        tf = p['tf'][i]

        nx = _fps(xyz, xyzt, npnt)
        nxt = jnp.swapaxes(nx, 1, 2)
        gidx = _knn(nx, xyzt, K, min(64, npnt))
        gsa = _gather_rows(sa_tab.reshape(B * n, _sa_td(c)), gidx)
        gsa = gsa.reshape(B, npnt * K, _sa_td(c))
        sa_w = [
            td[0]['w'][0:3, :], td[0]['w'][3:, :], _row(td[0]['b']),
            _row(td[0]['gamma']), _row(td[0]['beta']),
            td[1]['w'], _row(td[1]['b']),
            _row(td[1]['gamma']), _row(td[1]['beta']),
        ]
        pts_sa = _sa_post(gsa, nx, K, c, ch, sa_w)

        k_tf = min(K, npnt)
        q2, tab2 = _tf_pre(pts_sa, nx, tf['fc1']['w'], _row(tf['fc1']['b']),
                           tf['w_qs']['w'], tf['w_ks']['w'], tf['w_vs']['w'])
        idx2 = _knn(nx, nxt, k_tf, min(64, npnt))
        g2 = _gather_rows(tab2.reshape(B * npnt, 128), idx2)
        g2 = g2.reshape(B, npnt * k_tf, 128)
        points, sa_tab = _tf_post(g2, nx, q2, pts_sa, k_tf, tf_wlist(tf),
                                  i < 3)
        xyz, xyzt, n = nx, nxt, npnt

    f1, f2, f3 = p['fc2']
    res = _fc2(points.reshape(B * 8, ch), [
        f1['w'], _row(f1['b']), f2['w'], _row(f2['b']),
        f3['w'], _row(f3['b'])])
    return (res.reshape(B, 8, DM), xyz)


# speedup vs baseline: 1.3239x; 1.3239x over previous
import functools

import jax
import jax.numpy as jnp
from jax import lax
from jax.experimental import pallas as pl
from jax.experimental.pallas import tpu as pltpu
from jax.experimental.pallas import tpu_sc as plsc

B = 8
N0 = 2048
K = 16
DM = 32

_SC_CORES = 2
_SC_SUBCORES = 16
_SC_WORKERS = _SC_CORES * _SC_SUBCORES


def _sa_td(c):
    return ((16 + c + 127) // 128) * 128


def _lin(x, w, b=None):
    y = jnp.dot(x, w, preferred_element_type=jnp.float32)
    if b is not None:
        y = y + b
    return y


def _lin3(x, w, b=None):
    y = x[:, 0:1] * w[0:1, :] + x[:, 1:2] * w[1:2, :] + x[:, 2:3] * w[2:3, :]
    if b is not None:
        y = y + b
    return y


def _wspec(shape):
    nd = len(shape)
    return pl.BlockSpec(shape, lambda *_: (0,) * nd)


def _rep_rows(x, k):
    m, c = x.shape
    return jnp.broadcast_to(x[:, None, :], (m, k, c)).reshape(m * k, c)


def _gather_rows(table, idx):
    v, d = table.shape
    m = idx.shape[0]
    b_per_w = m // _SC_WORKERS
    ch = min(128, b_per_w)
    n_ch = b_per_w // ch
    gsz = max(1, min(n_ch, (384 * 1024) // (ch * d * 4)))
    while n_ch % gsz:
        gsz -= 1
    ngrp = n_ch // gsz
    mesh = plsc.VectorSubcoreMesh(core_axis_name="c", subcore_axis_name="s")

    @functools.partial(
        pl.kernel,
        out_type=jax.ShapeDtypeStruct((m, d), jnp.float32),
        mesh=mesh,
        scratch_types=[
            pltpu.VMEM((n_ch, ch), jnp.int32),
            pltpu.VMEM((gsz * ch, d), jnp.float32),
            pltpu.SemaphoreType.DMA,
        ],
    )
    def gk(table_hbm, idx_hbm, out_hbm, idx_v, rows_v, sem):
        wid = lax.axis_index("s") * _SC_CORES + lax.axis_index("c")
        base = wid * b_per_w
        pltpu.sync_copy(idx_hbm.at[pl.ds(wid * n_ch, n_ch)], idx_v)

        def group(gi, carry):
            handles = [
                pltpu.async_copy(
                    table_hbm.at[idx_v.at[gi * gsz + u]],
                    rows_v.at[pl.ds(u * ch, ch)], sem)
                for u in range(gsz)
            ]
            for h in handles:
                h.wait()
            pltpu.sync_copy(
                rows_v, out_hbm.at[pl.ds(base + gi * (gsz * ch), gsz * ch)])
            return carry

        if ngrp == 1:
            group(0, 0)
        else:
            lax.fori_loop(0, ngrp, group, 0)

    return gk(table, idx.reshape(m // ch, ch))


def _embed_body(x_ref, e1w, e1b, e2w, e2b, f1w, f1b, wq, wk, wv,
                h_ref, q_ref, tab_ref):
    xb = x_ref[0]
    h1 = jax.nn.relu(_lin3(xb, e1w[...], e1b[...]))
    h = _lin(h1, e2w[...], e2b[...])
    x1 = _lin(h, f1w[...], f1b[...])
    h_ref[0] = h
    q_ref[0] = _lin(x1, wq[...])
    tab_ref[0, :, 0:3] = xb
    tab_ref[0, :, 16:48] = _lin(x1, wk[...])
    tab_ref[0, :, 48:80] = _lin(x1, wv[...])


def _embed(x, e1w, e1b, e2w, e2b, f1w, f1b, wq, wk, wv):
    br = 256
    grid = (B, N0 // br)
    ws = [e1w, e1b, e2w, e2b, f1w, f1b, wq, wk, wv]
    return pl.pallas_call(
        _embed_body,
        grid=grid,
        in_specs=[pl.BlockSpec((1, br, 3), lambda b, r: (b, r, 0))]
        + [_wspec(w.shape) for w in ws],
        out_specs=[
            pl.BlockSpec((1, br, DM), lambda b, r: (b, r, 0)),
            pl.BlockSpec((1, br, DM), lambda b, r: (b, r, 0)),
            pl.BlockSpec((1, br, 128), lambda b, r: (b, r, 0)),
        ],
        out_shape=[
            jax.ShapeDtypeStruct((B, N0, DM), jnp.float32),
            jax.ShapeDtypeStruct((B, N0, DM), jnp.float32),
            jax.ShapeDtypeStruct((B, N0, 128), jnp.float32),
        ],
        compiler_params=pltpu.CompilerParams(
            dimension_semantics=("parallel", "arbitrary")),
    )(x, *ws)


def _knn_body(nd, k, src_ref, dstt_ref, idx_ref):
    b = pl.program_id(0)
    src = src_ref[0]
    br = src.shape[0]
    dt = dstt_ref[0]
    d = jnp.zeros((br, nd), jnp.float32)
    for c in range(3):
        diff = src[:, c:c + 1] - dt[c:c + 1, :]
        d = d + diff * diff
    iota = lax.broadcasted_iota(jnp.int32, (br, nd), 1)
    off = b * nd
    for j in range(k):
        jm = jnp.argmin(d, axis=1).astype(jnp.int32)[:, None]
        idx_ref[0, :, j:j + 1] = jm + off
        d = jnp.where(iota == jm, jnp.float32(jnp.inf), d)


def _knn(src, dstt, k, br):
    _, ns, _ = src.shape
    nd = dstt.shape[2]
    grid = (B, ns // br)
    out = pl.pallas_call(
        functools.partial(_knn_body, nd, k),
        grid=grid,
        in_specs=[
            pl.BlockSpec((1, br, 3), lambda b, r: (b, r, 0)),
            pl.BlockSpec((1, 3, nd), lambda b, r: (b, 0, 0)),
        ],
        out_specs=pl.BlockSpec((1, br, k), lambda b, r: (b, r, 0)),
        out_shape=jax.ShapeDtypeStruct((B, ns, k), jnp.int32),
        compiler_params=pltpu.CompilerParams(
            dimension_semantics=("parallel", "arbitrary")),
    )(src, dstt)
    return out.reshape(-1)


def _fps_body(n, npoint, xyz_ref, xyzt_ref, nx_ref):
    nl = n // 8
    flat = (lax.broadcasted_iota(jnp.int32, (8, nl), 0) * nl
            + lax.broadcasted_iota(jnp.int32, (8, nl), 1))

    def step(t, carry):
        dist, far = carry
        c = xyz_ref[0, pl.ds(far, 1), :]
        nx_ref[0, pl.ds(t, 1), :] = c
        d = jnp.zeros((8, nl), jnp.float32)
        for cc in range(3):
            diff = xyzt_ref[0, cc] - c[0, cc]
            d = d + diff * diff
        dist = jnp.minimum(dist, d)
        m = jnp.max(dist)
        far2 = jnp.min(jnp.where(dist >= m, flat, n))
        return dist, far2

    init = (jnp.full((8, nl), 1e10, jnp.float32), jnp.int32(0))
    lax.fori_loop(0, npoint, step, init)


def _fps(xyz, xyzt, npoint):
    n = xyz.shape[1]
    xyzr = xyzt.reshape(B, 3, 8, n // 8)
    return pl.pallas_call(
        functools.partial(_fps_body, n, npoint),
        grid=(B,),
        in_specs=[
            pl.BlockSpec((1, n, 3), lambda b: (b, 0, 0)),
            pl.BlockSpec((1, 3, 8, n // 8), lambda b: (b, 0, 0, 0)),
        ],
        out_specs=pl.BlockSpec((1, npoint, 3), lambda b: (b, 0, 0)),
        out_shape=jax.ShapeDtypeStruct((B, npoint, 3), jnp.float32),
        compiler_params=pltpu.CompilerParams(
            dimension_semantics=("parallel",)),
    )(xyz, xyzr)


def _tf_pre_body(pts_ref, xyz_ref, f1w, f1b, wq, wk, wv, q_ref, tab_ref):
    p = pts_ref[0]
    x1 = _lin(p, f1w[...], f1b[...])
    q_ref[0] = _lin(x1, wq[...])
    tab_ref[0, :, 0:3] = xyz_ref[0]
    tab_ref[0, :, 16:48] = _lin(x1, wk[...])
    tab_ref[0, :, 48:80] = _lin(x1, wv[...])


def _tf_pre(pts, xyz, f1w, f1b, wq, wk, wv):
    _, ns, c = pts.shape
    ws = [f1w, f1b, wq, wk, wv]
    return pl.pallas_call(
        _tf_pre_body,
        grid=(B,),
        in_specs=[
            pl.BlockSpec((1, ns, c), lambda b: (b, 0, 0)),
            pl.BlockSpec((1, ns, 3), lambda b: (b, 0, 0)),
        ] + [_wspec(w.shape) for w in ws],
        out_specs=[
            pl.BlockSpec((1, ns, DM), lambda b: (b, 0, 0)),
            pl.BlockSpec((1, ns, 128), lambda b: (b, 0, 0)),
        ],
        out_shape=[
            jax.ShapeDtypeStruct((B, ns, DM), jnp.float32),
            jax.ShapeDtypeStruct((B, ns, 128), jnp.float32),
        ],
        compiler_params=pltpu.CompilerParams(
            dimension_semantics=("parallel",)),
    )(pts, xyz, *ws)


def _tf_post_body(k, c, emit_tab, g_ref, xyz_ref, q_ref, pre_ref,
                  d1w, d1b, d2w, d2b, g1w, g1b, g2w, g2b, f2w, f2b,
                  *out_refs):
    out_ref = out_refs[0]
    g = g_ref[0]
    xyz = xyz_ref[0]
    br = xyz.shape[0]
    pos = _rep_rows(xyz, k) - g[:, 0:3]
    ph = jax.nn.relu(_lin3(pos, d1w[...], d1b[...]))
    pos_enc = _lin(ph, d2w[...], d2b[...])
    t = _rep_rows(q_ref[0], k) - g[:, 16:48] + pos_enc
    a1 = jax.nn.relu(_lin(t, g1w[...], g1b[...]))
    a2 = _lin(a1, g2w[...], g2b[...]) * jnp.float32(1.0 / (32.0 ** 0.5))
    a3 = a2.reshape(br, k, DM)
    mx = jnp.max(a3, axis=1)
    e = jnp.exp(a3 - mx[:, None, :])
    s = jnp.sum(e, axis=1)
    w3 = e / s[:, None, :]
    vp = (g[:, 48:80] + pos_enc).reshape(br, k, DM)
    res = jnp.sum(w3 * vp, axis=1)
    out = _lin(res, f2w[...], f2b[...]) + pre_ref[0]
    out_ref[0] = out
    if emit_tab:
        out_refs[1][0, :, 0:3] = xyz
        out_refs[1][0, :, 16:16 + c] = out


def _tf_post(g, xyz, q, pre, k, wlist, emit_tab):
    _, ns, c = pre.shape
    br = min(128, ns)
    grid = (B, ns // br)
    tab_d = _sa_td(c)
    out_specs = [pl.BlockSpec((1, br, c), lambda b, r: (b, r, 0))]
    out_shape = [jax.ShapeDtypeStruct((B, ns, c), jnp.float32)]
    if emit_tab:
        out_specs.append(pl.BlockSpec((1, br, tab_d), lambda b, r: (b, r, 0)))
        out_shape.append(jax.ShapeDtypeStruct((B, ns, tab_d), jnp.float32))
    outs = pl.pallas_call(
        functools.partial(_tf_post_body, k, c, emit_tab),
        grid=grid,
        in_specs=[
            pl.BlockSpec((1, br * k, 128), lambda b, r: (b, r, 0)),
            pl.BlockSpec((1, br, 3), lambda b, r: (b, r, 0)),
            pl.BlockSpec((1, br, DM), lambda b, r: (b, r, 0)),
            pl.BlockSpec((1, br, c), lambda b, r: (b, r, 0)),
        ] + [_wspec(w.shape) for w in wlist],
        out_specs=out_specs,
        out_shape=out_shape,
        compiler_params=pltpu.CompilerParams(
            dimension_semantics=("parallel", "arbitrary")),
    )(g, xyz, q, pre, *wlist)
    return outs if emit_tab else (outs[0], None)


def _sa_post_body(k, c, g_ref, nx_ref,
                  w1x, w1p, b1, ga1, be1, w2, b2, ga2, be2, out_ref):
    g = g_ref[0]
    nx = nx_ref[0]
    br = nx.shape[0]
    ch = out_ref.shape[2]
    inv = 1.0 / jnp.sqrt(jnp.float32(1.0 + 1e-5))
    xyzn = g[:, 0:3] - _rep_rows(nx, k)
    h = _lin3(xyzn, w1x[...]) + _lin(g[:, 16:16 + c], w1p[...]) + b1[...]
    h = jax.nn.relu(h * inv * ga1[...] + be1[...])
    h = _lin(h, w2[...], b2[...])
    h = jax.nn.relu(h * inv * ga2[...] + be2[...])
    out_ref[0] = jnp.max(h.reshape(br, k, ch), axis=1)


def _sa_post(g, nx, k, c, ch, wlist):
    npnt = nx.shape[1]
    br = min(128, npnt)
    grid = (B, npnt // br)
    tab_d = _sa_td(c)
    return pl.pallas_call(
        functools.partial(_sa_post_body, k, c),
        grid=grid,
        in_specs=[
            pl.BlockSpec((1, br * k, tab_d), lambda b, r: (b, r, 0)),
            pl.BlockSpec((1, br, 3), lambda b, r: (b, r, 0)),
        ] + [_wspec(w.shape) for w in wlist],
        out_specs=pl.BlockSpec((1, br, ch), lambda b, r: (b, r, 0)),
        out_shape=jax.ShapeDtypeStruct((B, npnt, ch), jnp.float32),
        compiler_params=pltpu.CompilerParams(
            dimension_semantics=("parallel", "arbitrary")),
    )(g, nx, *wlist)


def _fc2_body(p_ref, w1, b1, w2, b2, w3, b3, out_ref):
    r = jax.nn.relu(_lin(p_ref[...], w1[...], b1[...]))
    r = jax.nn.relu(_lin(r, w2[...], b2[...]))
    out_ref[...] = _lin(r, w3[...], b3[...])


def _fc2(pts, wlist):
    m = pts.shape[0]
    return pl.pallas_call(
        _fc2_body,
        out_shape=jax.ShapeDtypeStruct((m, DM), jnp.float32),
    )(pts, *wlist)


def _row(v):
    return v.reshape(1, -1)


def kernel(x, params):
    p = params
    xyz = x
    xyzt = jnp.swapaxes(xyz, 1, 2)

    e1, e2 = p['fc1']
    t1 = p['transformer1']
    h, q, tab = _embed(
        x, e1['w'], _row(e1['b']), e2['w'], _row(e2['b']),
        t1['fc1']['w'], _row(t1['fc1']['b']),
        t1['w_qs']['w'], t1['w_ks']['w'], t1['w_vs']['w'])

    def tf_wlist(tp):
        return [
            tp['fc_delta'][0]['w'], _row(tp['fc_delta'][0]['b']),
            tp['fc_delta'][1]['w'], _row(tp['fc_delta'][1]['b']),
            tp['fc_gamma'][0]['w'], _row(tp['fc_gamma'][0]['b']),
            tp['fc_gamma'][1]['w'], _row(tp['fc_gamma'][1]['b']),
            tp['fc2']['w'], _row(tp['fc2']['b']),
        ]

    n = N0
    idx = _knn(xyz, xyzt, K, 128)
    g = _gather_rows(tab.reshape(B * n, 128), idx).reshape(B, n * K, 128)
    points, sa_tab = _tf_post(g, xyz, q, h, K, tf_wlist(t1), True)

    for i in range(4):
        npnt = N0 // 4 ** (i + 1)
        ch = DM * 2 ** (i + 1)
        c = ch // 2
        td = p['td'][i]['mlps']
        tf = p['tf'][i]

        nx = _fps(xyz, xyzt, npnt)
        nxt = jnp.swapaxes(nx, 1, 2)
        gidx = _knn(nx, xyzt, K, min(128, npnt))
        gsa = _gather_rows(sa_tab.reshape(B * n, _sa_td(c)), gidx)
        gsa = gsa.reshape(B, npnt * K, _sa_td(c))
        sa_w = [
            td[0]['w'][0:3, :], td[0]['w'][3:, :], _row(td[0]['b']),
            _row(td[0]['gamma']), _row(td[0]['beta']),
            td[1]['w'], _row(td[1]['b']),
            _row(td[1]['gamma']), _row(td[1]['beta']),
        ]
        pts_sa = _sa_post(gsa, nx, K, c, ch, sa_w)

        k_tf = min(K, npnt)
        q2, tab2 = _tf_pre(pts_sa, nx, tf['fc1']['w'], _row(tf['fc1']['b']),
                           tf['w_qs']['w'], tf['w_ks']['w'], tf['w_vs']['w'])
        idx2 = _knn(nx, nxt, k_tf, min(128, npnt))
        g2 = _gather_rows(tab2.reshape(B * npnt, 128), idx2)
        g2 = g2.reshape(B, npnt * k_tf, 128)
        points, sa_tab = _tf_post(g2, nx, q2, pts_sa, k_tf, tf_wlist(tf),
                                  i < 3)
        xyz, xyzt, n = nx, nxt, npnt

    f1, f2, f3 = p['fc2']
    res = _fc2(points.reshape(B * 8, ch), [
        f1['w'], _row(f1['b']), f2['w'], _row(f2['b']),
        f3['w'], _row(f3['b'])])
    return (res.reshape(B, 8, DM), xyz)


# speedup vs baseline: 1.3537x; 1.0226x over previous
import functools

import jax
import jax.numpy as jnp
from jax import lax
from jax.experimental import pallas as pl
from jax.experimental.pallas import tpu as pltpu
from jax.experimental.pallas import tpu_sc as plsc

B = 8
N0 = 2048
K = 16
DM = 32

_SC_CORES = 2
_SC_SUBCORES = 16
_SC_WORKERS = _SC_CORES * _SC_SUBCORES


def _sa_td(c):
    return ((16 + c + 127) // 128) * 128


def _lin(x, w, b=None):
    y = jnp.dot(x, w, preferred_element_type=jnp.float32)
    if b is not None:
        y = y + b
    return y


def _lin3(x, w, b=None):
    y = x[:, 0:1] * w[0:1, :] + x[:, 1:2] * w[1:2, :] + x[:, 2:3] * w[2:3, :]
    if b is not None:
        y = y + b
    return y


def _wspec(shape):
    nd = len(shape)
    return pl.BlockSpec(shape, lambda *_: (0,) * nd)


def _rep_rows(x, k):
    m, c = x.shape
    return jnp.broadcast_to(x[:, None, :], (m, k, c)).reshape(m * k, c)


def _gather_rows(table, idx):
    v, d = table.shape
    m = idx.shape[0]
    b_per_w = m // _SC_WORKERS
    ch = min(128, b_per_w)
    n_ch = b_per_w // ch
    gsz = max(1, min(n_ch, (384 * 1024) // (ch * d * 4)))
    while n_ch % gsz:
        gsz -= 1
    ngrp = n_ch // gsz
    mesh = plsc.VectorSubcoreMesh(core_axis_name="c", subcore_axis_name="s")

    @functools.partial(
        pl.kernel,
        out_type=jax.ShapeDtypeStruct((m, d), jnp.float32),
        mesh=mesh,
        scratch_types=[
            pltpu.VMEM((n_ch, ch), jnp.int32),
            pltpu.VMEM((gsz * ch, d), jnp.float32),
            pltpu.SemaphoreType.DMA,
        ],
    )
    def gk(table_hbm, idx_hbm, out_hbm, idx_v, rows_v, sem):
        wid = lax.axis_index("s") * _SC_CORES + lax.axis_index("c")
        base = wid * b_per_w
        pltpu.sync_copy(idx_hbm.at[pl.ds(wid * n_ch, n_ch)], idx_v)

        def group(gi, carry):
            handles = [
                pltpu.async_copy(
                    table_hbm.at[idx_v.at[gi * gsz + u]],
                    rows_v.at[pl.ds(u * ch, ch)], sem)
                for u in range(gsz)
            ]
            for h in handles:
                h.wait()
            pltpu.sync_copy(
                rows_v, out_hbm.at[pl.ds(base + gi * (gsz * ch), gsz * ch)])
            return carry

        if ngrp == 1:
            group(0, 0)
        else:
            lax.fori_loop(0, ngrp, group, 0)

    return gk(table, idx.reshape(m // ch, ch))


def _embed_body(x_ref, e1w, e1b, e2w, e2b, f1w, f1b, wq, wk, wv,
                h_ref, q_ref, tab_ref):
    xb = x_ref[0]
    h1 = jax.nn.relu(_lin3(xb, e1w[...], e1b[...]))
    h = _lin(h1, e2w[...], e2b[...])
    x1 = _lin(h, f1w[...], f1b[...])
    h_ref[0] = h
    q_ref[0] = _lin(x1, wq[...])
    tab_ref[0, :, 0:3] = xb
    tab_ref[0, :, 16:48] = _lin(x1, wk[...])
    tab_ref[0, :, 48:80] = _lin(x1, wv[...])


def _embed(x, e1w, e1b, e2w, e2b, f1w, f1b, wq, wk, wv):
    br = 256
    grid = (B, N0 // br)
    ws = [e1w, e1b, e2w, e2b, f1w, f1b, wq, wk, wv]
    return pl.pallas_call(
        _embed_body,
        grid=grid,
        in_specs=[pl.BlockSpec((1, br, 3), lambda b, r: (b, r, 0))]
        + [_wspec(w.shape) for w in ws],
        out_specs=[
            pl.BlockSpec((1, br, DM), lambda b, r: (b, r, 0)),
            pl.BlockSpec((1, br, DM), lambda b, r: (b, r, 0)),
            pl.BlockSpec((1, br, 128), lambda b, r: (b, r, 0)),
        ],
        out_shape=[
            jax.ShapeDtypeStruct((B, N0, DM), jnp.float32),
            jax.ShapeDtypeStruct((B, N0, DM), jnp.float32),
            jax.ShapeDtypeStruct((B, N0, 128), jnp.float32),
        ],
        compiler_params=pltpu.CompilerParams(
            dimension_semantics=("parallel", "arbitrary")),
    )(x, *ws)


def _knn_body(nd, k, src_ref, dstt_ref, idx_ref):
    b = pl.program_id(0)
    src = src_ref[0]
    br = src.shape[0]
    dt = dstt_ref[0]
    d = jnp.zeros((br, nd), jnp.float32)
    for c in range(3):
        diff = src[:, c:c + 1] - dt[c:c + 1, :]
        d = d + diff * diff
    iota = lax.broadcasted_iota(jnp.int32, (br, nd), 1)
    off = b * nd
    for j in range(k):
        jm = jnp.argmin(d, axis=1).astype(jnp.int32)[:, None]
        idx_ref[0, :, j:j + 1] = jm + off
        d = jnp.where(iota == jm, jnp.float32(jnp.inf), d)


def _knn(src, dstt, k, br):
    _, ns, _ = src.shape
    nd = dstt.shape[2]
    grid = (B, ns // br)
    out = pl.pallas_call(
        functools.partial(_knn_body, nd, k),
        grid=grid,
        in_specs=[
            pl.BlockSpec((1, br, 3), lambda b, r: (b, r, 0)),
            pl.BlockSpec((1, 3, nd), lambda b, r: (b, 0, 0)),
        ],
        out_specs=pl.BlockSpec((1, br, k), lambda b, r: (b, r, 0)),
        out_shape=jax.ShapeDtypeStruct((B, ns, k), jnp.int32),
        compiler_params=pltpu.CompilerParams(
            dimension_semantics=("parallel", "arbitrary")),
    )(src, dstt)
    return out.reshape(-1)


def _fps_body(n, npoint, xyz_ref, xyzt_ref, nx_ref):
    nl = n // 8
    flat = (lax.broadcasted_iota(jnp.int32, (8, nl), 0) * nl
            + lax.broadcasted_iota(jnp.int32, (8, nl), 1))

    def step(t, carry):
        dist, far = carry
        c = xyz_ref[0, pl.ds(far, 1), :]
        nx_ref[0, pl.ds(t, 1), :] = c
        d = jnp.zeros((8, nl), jnp.float32)
        for cc in range(3):
            diff = xyzt_ref[0, cc] - c[0, cc]
            d = d + diff * diff
        dist = jnp.minimum(dist, d)
        m = jnp.max(dist)
        far2 = jnp.min(jnp.where(dist >= m, flat, n))
        return dist, far2

    init = (jnp.full((8, nl), 1e10, jnp.float32), jnp.int32(0))
    lax.fori_loop(0, npoint, step, init)


def _fps(xyz, xyzt, npoint):
    n = xyz.shape[1]
    xyzr = xyzt.reshape(B, 3, 8, n // 8)
    return pl.pallas_call(
        functools.partial(_fps_body, n, npoint),
        grid=(B,),
        in_specs=[
            pl.BlockSpec((1, n, 3), lambda b: (b, 0, 0)),
            pl.BlockSpec((1, 3, 8, n // 8), lambda b: (b, 0, 0, 0)),
        ],
        out_specs=pl.BlockSpec((1, npoint, 3), lambda b: (b, 0, 0)),
        out_shape=jax.ShapeDtypeStruct((B, npoint, 3), jnp.float32),
        compiler_params=pltpu.CompilerParams(
            dimension_semantics=("parallel",)),
    )(xyz, xyzr)


def _tf_pre_body(pts_ref, xyz_ref, f1w, f1b, wq, wk, wv, q_ref, tab_ref):
    p = pts_ref[0]
    x1 = _lin(p, f1w[...], f1b[...])
    q_ref[0] = _lin(x1, wq[...])
    tab_ref[0, :, 0:3] = xyz_ref[0]
    tab_ref[0, :, 16:48] = _lin(x1, wk[...])
    tab_ref[0, :, 48:80] = _lin(x1, wv[...])


def _tf_pre(pts, xyz, f1w, f1b, wq, wk, wv):
    _, ns, c = pts.shape
    ws = [f1w, f1b, wq, wk, wv]
    return pl.pallas_call(
        _tf_pre_body,
        grid=(B,),
        in_specs=[
            pl.BlockSpec((1, ns, c), lambda b: (b, 0, 0)),
            pl.BlockSpec((1, ns, 3), lambda b: (b, 0, 0)),
        ] + [_wspec(w.shape) for w in ws],
        out_specs=[
            pl.BlockSpec((1, ns, DM), lambda b: (b, 0, 0)),
            pl.BlockSpec((1, ns, 128), lambda b: (b, 0, 0)),
        ],
        out_shape=[
            jax.ShapeDtypeStruct((B, ns, DM), jnp.float32),
            jax.ShapeDtypeStruct((B, ns, 128), jnp.float32),
        ],
        compiler_params=pltpu.CompilerParams(
            dimension_semantics=("parallel",)),
    )(pts, xyz, *ws)


def _tf_post_body(k, c, emit_tab, g_ref, xyz_ref, q_ref, pre_ref,
                  d1w, d1b, d2w, d2b, g1w, g1b, g2w, g2b, f2w, f2b,
                  *out_refs):
    out_ref = out_refs[0]
    g = g_ref[0]
    xyz = xyz_ref[0]
    br = xyz.shape[0]
    pos = _rep_rows(xyz, k) - g[:, 0:3]
    ph = jax.nn.relu(_lin3(pos, d1w[...], d1b[...]))
    pos_enc = _lin(ph, d2w[...], d2b[...])
    t = _rep_rows(q_ref[0], k) - g[:, 16:48] + pos_enc
    a1 = jax.nn.relu(_lin(t, g1w[...], g1b[...]))
    a2 = _lin(a1, g2w[...], g2b[...]) * jnp.float32(1.0 / (32.0 ** 0.5))
    a3 = a2.reshape(br, k, DM)
    mx = jnp.max(a3, axis=1)
    e = jnp.exp(a3 - mx[:, None, :])
    s = jnp.sum(e, axis=1)
    w3 = e / s[:, None, :]
    vp = (g[:, 48:80] + pos_enc).reshape(br, k, DM)
    res = jnp.sum(w3 * vp, axis=1)
    out = _lin(res, f2w[...], f2b[...]) + pre_ref[0]
    out_ref[0] = out
    if emit_tab:
        out_refs[1][0, :, 0:3] = xyz
        out_refs[1][0, :, 16:16 + c] = out


def _tf_post(g, xyz, q, pre, k, wlist, emit_tab):
    _, ns, c = pre.shape
    br = min(128, ns)
    grid = (B, ns // br)
    tab_d = _sa_td(c)
    out_specs = [pl.BlockSpec((1, br, c), lambda b, r: (b, r, 0))]
    out_shape = [jax.ShapeDtypeStruct((B, ns, c), jnp.float32)]
    if emit_tab:
        out_specs.append(pl.BlockSpec((1, br, tab_d), lambda b, r: (b, r, 0)))
        out_shape.append(jax.ShapeDtypeStruct((B, ns, tab_d), jnp.float32))
    outs = pl.pallas_call(
        functools.partial(_tf_post_body, k, c, emit_tab),
        grid=grid,
        in_specs=[
            pl.BlockSpec((1, br * k, 128), lambda b, r: (b, r, 0)),
            pl.BlockSpec((1, br, 3), lambda b, r: (b, r, 0)),
            pl.BlockSpec((1, br, DM), lambda b, r: (b, r, 0)),
            pl.BlockSpec((1, br, c), lambda b, r: (b, r, 0)),
        ] + [_wspec(w.shape) for w in wlist],
        out_specs=out_specs,
        out_shape=out_shape,
        compiler_params=pltpu.CompilerParams(
            dimension_semantics=("parallel", "arbitrary")),
    )(g, xyz, q, pre, *wlist)
    return outs if emit_tab else (outs[0], None)


def _sa_post_body(k, c, g_ref, nx_ref,
                  w1x, w1p, b1, ga1, be1, w2, b2, ga2, be2, out_ref):
    g = g_ref[0]
    nx = nx_ref[0]
    br = nx.shape[0]
    ch = out_ref.shape[2]
    inv = 1.0 / jnp.sqrt(jnp.float32(1.0 + 1e-5))
    xyzn = g[:, 0:3] - _rep_rows(nx, k)
    h = _lin3(xyzn, w1x[...]) + _lin(g[:, 16:16 + c], w1p[...]) + b1[...]
    h = jax.nn.relu(h * inv * ga1[...] + be1[...])
    h = _lin(h, w2[...], b2[...])
    h = jax.nn.relu(h * inv * ga2[...] + be2[...])
    out_ref[0] = jnp.max(h.reshape(br, k, ch), axis=1)


def _sa_post(g, nx, k, c, ch, wlist):
    npnt = nx.shape[1]
    br = min(128, npnt)
    grid = (B, npnt // br)
    tab_d = _sa_td(c)
    return pl.pallas_call(
        functools.partial(_sa_post_body, k, c),
        grid=grid,
        in_specs=[
            pl.BlockSpec((1, br * k, tab_d), lambda b, r: (b, r, 0)),
            pl.BlockSpec((1, br, 3), lambda b, r: (b, r, 0)),
        ] + [_wspec(w.shape) for w in wlist],
        out_specs=pl.BlockSpec((1, br, ch), lambda b, r: (b, r, 0)),
        out_shape=jax.ShapeDtypeStruct((B, npnt, ch), jnp.float32),
        compiler_params=pltpu.CompilerParams(
            dimension_semantics=("parallel", "arbitrary")),
    )(g, nx, *wlist)


def _fc2_body(p_ref, w1, b1, w2, b2, w3, b3, out_ref):
    r = jax.nn.relu(_lin(p_ref[...], w1[...], b1[...]))
    r = jax.nn.relu(_lin(r, w2[...], b2[...]))
    out_ref[...] = _lin(r, w3[...], b3[...])


def _fc2(pts, wlist):
    m = pts.shape[0]
    return pl.pallas_call(
        _fc2_body,
        out_shape=jax.ShapeDtypeStruct((m, DM), jnp.float32),
    )(pts, *wlist)


def _row(v):
    return v.reshape(1, -1)


def kernel(x, params):
    p = params
    xyz = x
    xyzt = jnp.swapaxes(xyz, 1, 2)

    e1, e2 = p['fc1']
    t1 = p['transformer1']
    h, q, tab = _embed(
        x, e1['w'], _row(e1['b']), e2['w'], _row(e2['b']),
        t1['fc1']['w'], _row(t1['fc1']['b']),
        t1['w_qs']['w'], t1['w_ks']['w'], t1['w_vs']['w'])

    def tf_wlist(tp):
        return [
            tp['fc_delta'][0]['w'], _row(tp['fc_delta'][0]['b']),
            tp['fc_delta'][1]['w'], _row(tp['fc_delta'][1]['b']),
            tp['fc_gamma'][0]['w'], _row(tp['fc_gamma'][0]['b']),
            tp['fc_gamma'][1]['w'], _row(tp['fc_gamma'][1]['b']),
            tp['fc2']['w'], _row(tp['fc2']['b']),
        ]

    n = N0
    idx = _knn(xyz, xyzt, K, 256)
    g = _gather_rows(tab.reshape(B * n, 128), idx).reshape(B, n * K, 128)
    points, sa_tab = _tf_post(g, xyz, q, h, K, tf_wlist(t1), True)

    for i in range(4):
        npnt = N0 // 4 ** (i + 1)
        ch = DM * 2 ** (i + 1)
        c = ch // 2
        td = p['td'][i]['mlps']
        tf = p['tf'][i]

        nx = _fps(xyz, xyzt, npnt)
        nxt = jnp.swapaxes(nx, 1, 2)
        gidx = _knn(nx, xyzt, K, min(256, npnt))
        gsa = _gather_rows(sa_tab.reshape(B * n, _sa_td(c)), gidx)
        gsa = gsa.reshape(B, npnt * K, _sa_td(c))
        sa_w = [
            td[0]['w'][0:3, :], td[0]['w'][3:, :], _row(td[0]['b']),
            _row(td[0]['gamma']), _row(td[0]['beta']),
            td[1]['w'], _row(td[1]['b']),
            _row(td[1]['gamma']), _row(td[1]['beta']),
        ]
        pts_sa = _sa_post(gsa, nx, K, c, ch, sa_w)

        k_tf = min(K, npnt)
        q2, tab2 = _tf_pre(pts_sa, nx, tf['fc1']['w'], _row(tf['fc1']['b']),
                           tf['w_qs']['w'], tf['w_ks']['w'], tf['w_vs']['w'])
        idx2 = _knn(nx, nxt, k_tf, min(256, npnt))
        g2 = _gather_rows(tab2.reshape(B * npnt, 128), idx2)
        g2 = g2.reshape(B, npnt * k_tf, 128)
        points, sa_tab = _tf_post(g2, nx, q2, pts_sa, k_tf, tf_wlist(tf),
                                  i < 3)
        xyz, xyzt, n = nx, nxt, npnt

    f1, f2, f3 = p['fc2']
    res = _fc2(points.reshape(B * 8, ch), [
        f1['w'], _row(f1['b']), f2['w'], _row(f2['b']),
        f3['w'], _row(f3['b'])])
    return (res.reshape(B, 8, DM), xyz)


# speedup vs baseline: 1.3601x; 1.0047x over previous
import functools

import jax
import jax.numpy as jnp
from jax import lax
from jax.experimental import pallas as pl
from jax.experimental.pallas import tpu as pltpu
from jax.experimental.pallas import tpu_sc as plsc

B = 8
N0 = 2048
K = 16
DM = 32

_SC_CORES = 2
_SC_SUBCORES = 16
_SC_WORKERS = _SC_CORES * _SC_SUBCORES


def _sa_td(c):
    return ((16 + c + 127) // 128) * 128


def _lin(x, w, b=None):
    y = jnp.dot(x, w, preferred_element_type=jnp.float32)
    if b is not None:
        y = y + b
    return y


def _lin3(x, w, b=None):
    y = x[:, 0:1] * w[0:1, :] + x[:, 1:2] * w[1:2, :] + x[:, 2:3] * w[2:3, :]
    if b is not None:
        y = y + b
    return y


def _wspec(shape):
    nd = len(shape)
    return pl.BlockSpec(shape, lambda *_: (0,) * nd)


def _rep_rows(x, k):
    m, c = x.shape
    return jnp.broadcast_to(x[:, None, :], (m, k, c)).reshape(m * k, c)


def _gather_rows(table, idx):
    v, d = table.shape
    m = idx.shape[0]
    b_per_w = m // _SC_WORKERS
    ch = min(128, b_per_w)
    n_ch = b_per_w // ch
    gsz = max(1, min(n_ch, (384 * 1024) // (ch * d * 4)))
    while n_ch % gsz:
        gsz -= 1
    ngrp = n_ch // gsz
    mesh = plsc.VectorSubcoreMesh(core_axis_name="c", subcore_axis_name="s")

    @functools.partial(
        pl.kernel,
        out_type=jax.ShapeDtypeStruct((m, d), jnp.float32),
        mesh=mesh,
        scratch_types=[
            pltpu.VMEM((n_ch, ch), jnp.int32),
            pltpu.VMEM((gsz * ch, d), jnp.float32),
            pltpu.SemaphoreType.DMA,
        ],
    )
    def gk(table_hbm, idx_hbm, out_hbm, idx_v, rows_v, sem):
        wid = lax.axis_index("s") * _SC_CORES + lax.axis_index("c")
        base = wid * b_per_w
        pltpu.sync_copy(idx_hbm.at[pl.ds(wid * n_ch, n_ch)], idx_v)

        def group(gi, carry):
            handles = [
                pltpu.async_copy(
                    table_hbm.at[idx_v.at[gi * gsz + u]],
                    rows_v.at[pl.ds(u * ch, ch)], sem)
                for u in range(gsz)
            ]
            for h in handles:
                h.wait()
            pltpu.sync_copy(
                rows_v, out_hbm.at[pl.ds(base + gi * (gsz * ch), gsz * ch)])
            return carry

        if ngrp == 1:
            group(0, 0)
        else:
            lax.fori_loop(0, ngrp, group, 0)

    return gk(table, idx.reshape(m // ch, ch))


def _embed_body(x_ref, e1w, e1b, e2w, e2b, f1w, f1b, wq, wk, wv,
                h_ref, q_ref, tab_ref):
    xb = x_ref[0]
    h1 = jax.nn.relu(_lin3(xb, e1w[...], e1b[...]))
    h = _lin(h1, e2w[...], e2b[...])
    x1 = _lin(h, f1w[...], f1b[...])
    h_ref[0] = h
    q_ref[0] = _lin(x1, wq[...])
    tab_ref[0, :, 0:3] = xb
    tab_ref[0, :, 16:48] = _lin(x1, wk[...])
    tab_ref[0, :, 48:80] = _lin(x1, wv[...])


def _embed(x, e1w, e1b, e2w, e2b, f1w, f1b, wq, wk, wv):
    br = 256
    grid = (B, N0 // br)
    ws = [e1w, e1b, e2w, e2b, f1w, f1b, wq, wk, wv]
    return pl.pallas_call(
        _embed_body,
        grid=grid,
        in_specs=[pl.BlockSpec((1, br, 3), lambda b, r: (b, r, 0))]
        + [_wspec(w.shape) for w in ws],
        out_specs=[
            pl.BlockSpec((1, br, DM), lambda b, r: (b, r, 0)),
            pl.BlockSpec((1, br, DM), lambda b, r: (b, r, 0)),
            pl.BlockSpec((1, br, 128), lambda b, r: (b, r, 0)),
        ],
        out_shape=[
            jax.ShapeDtypeStruct((B, N0, DM), jnp.float32),
            jax.ShapeDtypeStruct((B, N0, DM), jnp.float32),
            jax.ShapeDtypeStruct((B, N0, 128), jnp.float32),
        ],
        compiler_params=pltpu.CompilerParams(
            dimension_semantics=("parallel", "arbitrary")),
    )(x, *ws)


def _knn_body(nd, k, src_ref, dstt_ref, idx_ref):
    b = pl.program_id(0)
    src = src_ref[0]
    br = src.shape[0]
    dt = dstt_ref[0]
    d = jnp.zeros((br, nd), jnp.float32)
    for c in range(3):
        diff = src[:, c:c + 1] - dt[c:c + 1, :]
        d = d + diff * diff
    iota = lax.broadcasted_iota(jnp.int32, (br, nd), 1)
    off = b * nd
    for j in range(k):
        jm = jnp.argmin(d, axis=1).astype(jnp.int32)[:, None]
        idx_ref[0, :, j:j + 1] = jm + off
        d = jnp.where(iota == jm, jnp.float32(jnp.inf), d)


def _knn(src, dstt, k, br):
    _, ns, _ = src.shape
    nd = dstt.shape[2]
    grid = (B, ns // br)
    out = pl.pallas_call(
        functools.partial(_knn_body, nd, k),
        grid=grid,
        in_specs=[
            pl.BlockSpec((1, br, 3), lambda b, r: (b, r, 0)),
            pl.BlockSpec((1, 3, nd), lambda b, r: (b, 0, 0)),
        ],
        out_specs=pl.BlockSpec((1, br, k), lambda b, r: (b, r, 0)),
        out_shape=jax.ShapeDtypeStruct((B, ns, k), jnp.int32),
        compiler_params=pltpu.CompilerParams(
            dimension_semantics=("parallel", "arbitrary")),
    )(src, dstt)
    return out.reshape(-1)


def _fps_body(n, npoint, xyz_ref, xyzt_ref, nx_ref):
    nl = n // 8
    flat = (lax.broadcasted_iota(jnp.int32, (8, nl), 0) * nl
            + lax.broadcasted_iota(jnp.int32, (8, nl), 1))

    def step(t, carry):
        dist, far = carry
        c = xyz_ref[0, pl.ds(far, 1), :]
        nx_ref[0, pl.ds(t, 1), :] = c
        d = jnp.zeros((8, nl), jnp.float32)
        for cc in range(3):
            diff = xyzt_ref[0, cc] - c[0, cc]
            d = d + diff * diff
        dist = jnp.minimum(dist, d)
        m = jnp.max(dist)
        far2 = jnp.min(jnp.where(dist >= m, flat, n))
        return dist, far2

    init = (jnp.full((8, nl), 1e10, jnp.float32), jnp.int32(0))
    lax.fori_loop(0, npoint, step, init, unroll=2)


def _fps(xyz, xyzt, npoint):
    n = xyz.shape[1]
    xyzr = xyzt.reshape(B, 3, 8, n // 8)
    return pl.pallas_call(
        functools.partial(_fps_body, n, npoint),
        grid=(B,),
        in_specs=[
            pl.BlockSpec((1, n, 3), lambda b: (b, 0, 0)),
            pl.BlockSpec((1, 3, 8, n // 8), lambda b: (b, 0, 0, 0)),
        ],
        out_specs=pl.BlockSpec((1, npoint, 3), lambda b: (b, 0, 0)),
        out_shape=jax.ShapeDtypeStruct((B, npoint, 3), jnp.float32),
        compiler_params=pltpu.CompilerParams(
            dimension_semantics=("parallel",)),
    )(xyz, xyzr)


def _tf_pre_body(pts_ref, xyz_ref, f1w, f1b, wq, wk, wv, q_ref, tab_ref):
    p = pts_ref[0]
    x1 = _lin(p, f1w[...], f1b[...])
    q_ref[0] = _lin(x1, wq[...])
    tab_ref[0, :, 0:3] = xyz_ref[0]
    tab_ref[0, :, 16:48] = _lin(x1, wk[...])
    tab_ref[0, :, 48:80] = _lin(x1, wv[...])


def _tf_pre(pts, xyz, f1w, f1b, wq, wk, wv):
    _, ns, c = pts.shape
    ws = [f1w, f1b, wq, wk, wv]
    return pl.pallas_call(
        _tf_pre_body,
        grid=(B,),
        in_specs=[
            pl.BlockSpec((1, ns, c), lambda b: (b, 0, 0)),
            pl.BlockSpec((1, ns, 3), lambda b: (b, 0, 0)),
        ] + [_wspec(w.shape) for w in ws],
        out_specs=[
            pl.BlockSpec((1, ns, DM), lambda b: (b, 0, 0)),
            pl.BlockSpec((1, ns, 128), lambda b: (b, 0, 0)),
        ],
        out_shape=[
            jax.ShapeDtypeStruct((B, ns, DM), jnp.float32),
            jax.ShapeDtypeStruct((B, ns, 128), jnp.float32),
        ],
        compiler_params=pltpu.CompilerParams(
            dimension_semantics=("parallel",)),
    )(pts, xyz, *ws)


def _tf_post_body(k, c, emit_tab, g_ref, xyz_ref, q_ref, pre_ref,
                  d1w, d1b, d2w, d2b, g1w, g1b, g2w, g2b, f2w, f2b,
                  *out_refs):
    out_ref = out_refs[0]
    g = g_ref[0]
    xyz = xyz_ref[0]
    br = xyz.shape[0]
    pos = _rep_rows(xyz, k) - g[:, 0:3]
    ph = jax.nn.relu(_lin3(pos, d1w[...], d1b[...]))
    pos_enc = _lin(ph, d2w[...], d2b[...])
    t = _rep_rows(q_ref[0], k) - g[:, 16:48] + pos_enc
    a1 = jax.nn.relu(_lin(t, g1w[...], g1b[...]))
    a2 = _lin(a1, g2w[...], g2b[...]) * jnp.float32(1.0 / (32.0 ** 0.5))
    a3 = a2.reshape(br, k, DM)
    mx = jnp.max(a3, axis=1)
    e = jnp.exp(a3 - mx[:, None, :])
    s = jnp.sum(e, axis=1)
    w3 = e / s[:, None, :]
    vp = (g[:, 48:80] + pos_enc).reshape(br, k, DM)
    res = jnp.sum(w3 * vp, axis=1)
    out = _lin(res, f2w[...], f2b[...]) + pre_ref[0]
    out_ref[0] = out
    if emit_tab:
        out_refs[1][0, :, 0:3] = xyz
        out_refs[1][0, :, 16:16 + c] = out


def _tf_post(g, xyz, q, pre, k, wlist, emit_tab):
    _, ns, c = pre.shape
    br = min(256, ns)
    grid = (B, ns // br)
    tab_d = _sa_td(c)
    out_specs = [pl.BlockSpec((1, br, c), lambda b, r: (b, r, 0))]
    out_shape = [jax.ShapeDtypeStruct((B, ns, c), jnp.float32)]
    if emit_tab:
        out_specs.append(pl.BlockSpec((1, br, tab_d), lambda b, r: (b, r, 0)))
        out_shape.append(jax.ShapeDtypeStruct((B, ns, tab_d), jnp.float32))
    outs = pl.pallas_call(
        functools.partial(_tf_post_body, k, c, emit_tab),
        grid=grid,
        in_specs=[
            pl.BlockSpec((1, br * k, 128), lambda b, r: (b, r, 0)),
            pl.BlockSpec((1, br, 3), lambda b, r: (b, r, 0)),
            pl.BlockSpec((1, br, DM), lambda b, r: (b, r, 0)),
            pl.BlockSpec((1, br, c), lambda b, r: (b, r, 0)),
        ] + [_wspec(w.shape) for w in wlist],
        out_specs=out_specs,
        out_shape=out_shape,
        compiler_params=pltpu.CompilerParams(
            dimension_semantics=("parallel", "arbitrary")),
    )(g, xyz, q, pre, *wlist)
    return outs if emit_tab else (outs[0], None)


def _sa_post_body(k, c, g_ref, nx_ref,
                  w1x, w1p, b1, ga1, be1, w2, b2, ga2, be2, out_ref):
    g = g_ref[0]
    nx = nx_ref[0]
    br = nx.shape[0]
    ch = out_ref.shape[2]
    inv = 1.0 / jnp.sqrt(jnp.float32(1.0 + 1e-5))
    xyzn = g[:, 0:3] - _rep_rows(nx, k)
    h = _lin3(xyzn, w1x[...]) + _lin(g[:, 16:16 + c], w1p[...]) + b1[...]
    h = jax.nn.relu(h * inv * ga1[...] + be1[...])
    h = _lin(h, w2[...], b2[...])
    h = jax.nn.relu(h * inv * ga2[...] + be2[...])
    out_ref[0] = jnp.max(h.reshape(br, k, ch), axis=1)


def _sa_post(g, nx, k, c, ch, wlist):
    npnt = nx.shape[1]
    br = min(128, npnt)
    grid = (B, npnt // br)
    tab_d = _sa_td(c)
    return pl.pallas_call(
        functools.partial(_sa_post_body, k, c),
        grid=grid,
        in_specs=[
            pl.BlockSpec((1, br * k, tab_d), lambda b, r: (b, r, 0)),
            pl.BlockSpec((1, br, 3), lambda b, r: (b, r, 0)),
        ] + [_wspec(w.shape) for w in wlist],
        out_specs=pl.BlockSpec((1, br, ch), lambda b, r: (b, r, 0)),
        out_shape=jax.ShapeDtypeStruct((B, npnt, ch), jnp.float32),
        compiler_params=pltpu.CompilerParams(
            dimension_semantics=("parallel", "arbitrary")),
    )(g, nx, *wlist)


def _fc2_body(p_ref, w1, b1, w2, b2, w3, b3, out_ref):
    r = jax.nn.relu(_lin(p_ref[...], w1[...], b1[...]))
    r = jax.nn.relu(_lin(r, w2[...], b2[...]))
    out_ref[...] = _lin(r, w3[...], b3[...])


def _fc2(pts, wlist):
    m = pts.shape[0]
    return pl.pallas_call(
        _fc2_body,
        out_shape=jax.ShapeDtypeStruct((m, DM), jnp.float32),
    )(pts, *wlist)


def _row(v):
    return v.reshape(1, -1)


def kernel(x, params):
    p = params
    xyz = x
    xyzt = jnp.swapaxes(xyz, 1, 2)

    e1, e2 = p['fc1']
    t1 = p['transformer1']
    h, q, tab = _embed(
        x, e1['w'], _row(e1['b']), e2['w'], _row(e2['b']),
        t1['fc1']['w'], _row(t1['fc1']['b']),
        t1['w_qs']['w'], t1['w_ks']['w'], t1['w_vs']['w'])

    def tf_wlist(tp):
        return [
            tp['fc_delta'][0]['w'], _row(tp['fc_delta'][0]['b']),
            tp['fc_delta'][1]['w'], _row(tp['fc_delta'][1]['b']),
            tp['fc_gamma'][0]['w'], _row(tp['fc_gamma'][0]['b']),
            tp['fc_gamma'][1]['w'], _row(tp['fc_gamma'][1]['b']),
            tp['fc2']['w'], _row(tp['fc2']['b']),
        ]

    n = N0
    idx = _knn(xyz, xyzt, K, 256)
    g = _gather_rows(tab.reshape(B * n, 128), idx).reshape(B, n * K, 128)
    points, sa_tab = _tf_post(g, xyz, q, h, K, tf_wlist(t1), True)

    for i in range(4):
        npnt = N0 // 4 ** (i + 1)
        ch = DM * 2 ** (i + 1)
        c = ch // 2
        td = p['td'][i]['mlps']
        tf = p['tf'][i]

        nx = _fps(xyz, xyzt, npnt)
        nxt = jnp.swapaxes(nx, 1, 2)
        gidx = _knn(nx, xyzt, K, min(256, npnt))
        gsa = _gather_rows(sa_tab.reshape(B * n, _sa_td(c)), gidx)
        gsa = gsa.reshape(B, npnt * K, _sa_td(c))
        sa_w = [
            td[0]['w'][0:3, :], td[0]['w'][3:, :], _row(td[0]['b']),
            _row(td[0]['gamma']), _row(td[0]['beta']),
            td[1]['w'], _row(td[1]['b']),
            _row(td[1]['gamma']), _row(td[1]['beta']),
        ]
        pts_sa = _sa_post(gsa, nx, K, c, ch, sa_w)

        k_tf = min(K, npnt)
        q2, tab2 = _tf_pre(pts_sa, nx, tf['fc1']['w'], _row(tf['fc1']['b']),
                           tf['w_qs']['w'], tf['w_ks']['w'], tf['w_vs']['w'])
        idx2 = _knn(nx, nxt, k_tf, min(256, npnt))
        g2 = _gather_rows(tab2.reshape(B * npnt, 128), idx2)
        g2 = g2.reshape(B, npnt * k_tf, 128)
        points, sa_tab = _tf_post(g2, nx, q2, pts_sa, k_tf, tf_wlist(tf),
                                  i < 3)
        xyz, xyzt, n = nx, nxt, npnt

    f1, f2, f3 = p['fc2']
    res = _fc2(points.reshape(B * 8, ch), [
        f1['w'], _row(f1['b']), f2['w'], _row(f2['b']),
        f3['w'], _row(f3['b'])])
    return (res.reshape(B, 8, DM), xyz)
